# baseline (speedup 1.0000x reference)
import functools

import jax
import jax.numpy as jnp
from jax import lax
from jax.experimental import pallas as pl
from jax.experimental.pallas import tpu as pltpu
from jax.experimental.pallas import tpu_sc as plsc

N = 10000
E = 320000
D = 128
C = 4
H = 32

NC = 2
NS = 16
NW = NC * NS
CHUNK = 128
KPW = -(-E // (NW * CHUNK))
E_PAD = NW * KPW * CHUNK
N_PAD = 10240
W128 = 128

Q = 4
QROWS = N_PAD // Q
QPAD = QROWS + 8

BE = 2528
N_EBLK = E_PAD // BE
NB = 512
N_NBLK = N_PAD // NB
NBQ = QROWS // NB

_f32 = jnp.float32


def _sc_mesh():
    return plsc.VectorSubcoreMesh(core_axis_name="c", subcore_axis_name="s")


def _gather_body(n_idx, *refs):
    h_hbm = refs[0]
    idx_hbm = refs[1:1 + n_idx]
    out_hbm = refs[1 + n_idx:1 + 2 * n_idx]
    idxv = refs[1 + 2 * n_idx:1 + 3 * n_idx]
    rows = refs[1 + 3 * n_idx:1 + 4 * n_idx]
    sem = refs[1 + 4 * n_idx]

    c = lax.axis_index("c")
    s = lax.axis_index("s")
    wid = c * NS + s
    for k in range(n_idx):
        pltpu.sync_copy(idx_hbm[k].at[wid], idxv[k])

    def chunk(j, carry):
        row0 = (wid * KPW + j) * CHUNK
        for k in range(n_idx):
            pltpu.async_copy(h_hbm.at[idxv[k].at[j]], rows[k], sem).wait()
            pltpu.sync_copy(rows[k], out_hbm[k].at[pl.ds(row0, CHUNK)])
        return carry

    lax.fori_loop(0, KPW, chunk, 0)


def _make_gather(n_idx):
    scratch = ([pltpu.VMEM((KPW, CHUNK), jnp.int32) for _ in range(n_idx)]
               + [pltpu.VMEM((CHUNK, W128), _f32) for _ in range(n_idx)]
               + [pltpu.SemaphoreType.DMA])
    out_type = [jax.ShapeDtypeStruct((E_PAD, W128), _f32)
                for _ in range(n_idx)]
    if n_idx == 1:
        out_type = out_type[0]
    return pl.kernel(
        functools.partial(_gather_body, n_idx),
        out_type=out_type,
        mesh=_sc_mesh(),
        scratch_types=scratch,
        name=f"sc_gather{n_idx}",
    )


ACC_COLS = (QROWS // 128) * H

_LANE = None


def _scatter_body(msg_hbm, dst_hbm, out_hbm, acc, didx, mrows):
    c = lax.axis_index("c")
    s = lax.axis_index("s")
    wid = c * NS + s
    pltpu.sync_copy(dst_hbm.at[wid], didx)
    lane = lax.iota(jnp.int32, 16)

    for q in range(Q):
        qbase = q * QROWS

        def zrow(r, carry):
            for g in range(ACC_COLS // 16):
                acc[r, pl.ds(g * 16, 16)] = jnp.zeros((16,), _f32)
            return carry

        lax.fori_loop(0, 128, zrow, 0)

        def chunk(j, carry):
            row0 = (wid * KPW + j) * CHUNK
            pltpu.sync_copy(msg_hbm.at[pl.ds(row0, CHUNK)], mrows)
            for g in range(CHUNK // 16):
                v = didx[j, pl.ds(g * 16, 16)]
                rel = v - qbase
                ok = (rel >= 0) & (rel < QROWS)
                relc = jnp.where(ok, rel, 0)
                row = relc & 127
                colbase = (relc >> 7) << 5
                evec = lane + (g * 16)
                for f in range(H):
                    fvec = jnp.full((16,), f, jnp.int32)
                    val = plsc.load_gather(mrows, [evec, fvec])
                    plsc.addupdate_scatter(acc, [row, colbase + fvec], val,
                                           mask=ok)
            return carry

        lax.fori_loop(0, KPW, chunk, 0)
        pltpu.sync_copy(acc,
                        out_hbm.at[pl.ds((wid * Q + q) * 128, 128)])


def _make_scatter():
    return pl.kernel(
        _scatter_body,
        out_type=jax.ShapeDtypeStruct((NW * Q * 128, ACC_COLS), _f32),
        mesh=_sc_mesh(),
        scratch_types=[
            pltpu.VMEM((128, ACC_COLS), _f32),
            pltpu.VMEM((KPW, CHUNK), jnp.int32),
            pltpu.VMEM((CHUNK, H), _f32),
        ],
        name="sc_scatter_add",
        compiler_params=pltpu.CompilerParams(needs_layout_passes=False),
    )


def _dot(a, b):
    return jnp.dot(a, b, preferred_element_type=_f32)


def _fold_enc(encW, encb_flat, edgeW, edgeb):
    M = jnp.concatenate(
        [_dot(encW[c:c + 1, :], edgeW[c * H:(c + 1) * H, :]) for c in range(C)],
        axis=0)
    cb = _dot(encb_flat, edgeW) + edgeb
    return M, cb


def _node_proj_body(x_ref, w_ref, b_ref, out_ref):
    h = _dot(x_ref[...], w_ref[...]) + b_ref[...]
    out_ref[:, :H] = h
    out_ref[:, H:] = jnp.zeros((NB, W128 - H), _f32)


def _node_proj(x_pad, node_W, node_b):
    return pl.pallas_call(
        _node_proj_body,
        grid=(N_NBLK,),
        in_specs=[
            pl.BlockSpec((NB, D), lambda i: (i, 0)),
            pl.BlockSpec((D, H), lambda i: (0, 0)),
            pl.BlockSpec((1, H), lambda i: (0, 0)),
        ],
        out_specs=pl.BlockSpec((NB, W128), lambda i: (i, 0)),
        out_shape=jax.ShapeDtypeStruct((N_PAD, W128), _f32),
    )(x_pad, node_W, node_b)


def _msg0_body(hs_ref, attr_ref, encW_ref, encb_ref, edgeW_ref, edgeb_ref,
               msg_ref):
    M, cb = _fold_enc(encW_ref[...], encb_ref[...], edgeW_ref[...],
                      edgeb_ref[...])
    ea0 = _dot(attr_ref[...], M) + cb
    msg_ref[...] = jnp.maximum(hs_ref[:, :H] + ea0, 0.0)


def _msg0(hs, attr, encW, encb_flat, edgeW, edgeb):
    full = lambda shape: pl.BlockSpec(shape, lambda i: (0, 0))
    return pl.pallas_call(
        _msg0_body,
        grid=(N_EBLK,),
        in_specs=[
            pl.BlockSpec((BE, W128), lambda i: (i, 0)),
            pl.BlockSpec((BE, C), lambda i: (i, 0)),
            full((C, H)), full((1, C * H)), full((C * H, H)), full((1, H)),
        ],
        out_specs=pl.BlockSpec((BE, H), lambda i: (i, 0)),
        out_shape=jax.ShapeDtypeStruct((E_PAD, H), _f32),
    )(hs, attr, encW, encb_flat, edgeW, edgeb)


def _node_upd_body(h_ref, ag_ref, w1_ref, b1_ref, w2_ref, b2_ref, eps_ref,
                   out_ref):
    h = h_ref[:, :H]
    agsum = jnp.sum(ag_ref[...], axis=0)[0]
    aggr = jnp.concatenate(
        [agsum[:, k * H:(k + 1) * H] for k in range(NB // 128)], axis=0)
    out = (1.0 + eps_ref[0, 0]) * h + aggr
    out = _dot(jnp.maximum(_dot(out, w1_ref[...]) + b1_ref[...], 0.0),
               w2_ref[...]) + b2_ref[...]
    out_ref[:, :H] = (h + jnp.maximum(out, 0.0)) * 0.5
    out_ref[:, H:] = jnp.zeros((NB, W128 - H), _f32)


def _node_update(h, aggr_p, w1, b1, w2, b2, eps_l):
    full = lambda shape: pl.BlockSpec(shape, lambda i: (0, 0))
    return pl.pallas_call(
        _node_upd_body,
        grid=(N_NBLK,),
        in_specs=[
            pl.BlockSpec((NB, W128), lambda i: (i, 0)),
            pl.BlockSpec((NW, 1, 128, 128),
                         lambda i: (0, i // NBQ, 0, i % NBQ)),
            full((H, H)), full((1, H)), full((H, H)), full((1, H)),
            pl.BlockSpec(memory_space=pltpu.SMEM),
        ],
        out_specs=pl.BlockSpec((NB, W128), lambda i: (i, 0)),
        out_shape=jax.ShapeDtypeStruct((N_PAD, W128), _f32),
    )(h, aggr_p, w1, b1, w2, b2, eps_l)


def _edge0_body(hs_ref, hd_ref, attr_ref, encW_ref, encb_ref, edgeW_ref,
                edgeb_ref, w1_ref, b1_ref, w2_ref, b2_ref,
                ea1_ref, msg1_ref):
    M, cb = _fold_enc(encW_ref[...], encb_ref[...], edgeW_ref[...],
                      edgeb_ref[...])
    ea0 = _dot(attr_ref[...], M) + cb
    hs = hs_ref[:, :H]
    w1 = w1_ref[...]
    pre = (_dot(hs, w1[0:H, :]) + _dot(hd_ref[:, :H], w1[H:2 * H, :])
           + _dot(ea0, w1[2 * H:3 * H, :]) + b1_ref[...])
    upd = _dot(jnp.maximum(pre, 0.0), w2_ref[...]) + b2_ref[...]
    ea1 = ea0 + 0.5 * upd
    ea1_ref[...] = ea1
    msg1_ref[...] = jnp.maximum(hs + ea1, 0.0)


def _edge0(hs, hd, attr, encW, encb_flat, edgeW, edgeb, w1, b1, w2, b2):
    full = lambda shape: pl.BlockSpec(shape, lambda i: (0, 0))
    eb = pl.BlockSpec((BE, H), lambda i: (i, 0))
    ebw = pl.BlockSpec((BE, W128), lambda i: (i, 0))
    return pl.pallas_call(
        _edge0_body,
        grid=(N_EBLK,),
        in_specs=[
            ebw, ebw,
            pl.BlockSpec((BE, C), lambda i: (i, 0)),
            full((C, H)), full((1, C * H)), full((C * H, H)), full((1, H)),
            full((3 * H, H)), full((1, H)), full((H, H)), full((1, H)),
        ],
        out_specs=[eb, eb],
        out_shape=[jax.ShapeDtypeStruct((E_PAD, H), _f32),
                   jax.ShapeDtypeStruct((E_PAD, H), _f32)],
    )(hs, hd, attr, encW, encb_flat, edgeW, edgeb, w1, b1, w2, b2)


def _edge1_cls_body(hs_ref, hd_ref, ea_ref, w1_ref, b1_ref, w2_ref, b2_ref,
                    cw1_ref, cb1_ref, cw2_ref, cb2_ref, out_ref):
    hs = hs_ref[:, :H]
    hd = hd_ref[:, :H]
    ea = ea_ref[...]
    w1 = w1_ref[...]
    pre = (_dot(hs, w1[0:H, :]) + _dot(hd, w1[H:2 * H, :])
           + _dot(ea, w1[2 * H:3 * H, :]) + b1_ref[...])
    upd = _dot(jnp.maximum(pre, 0.0), w2_ref[...]) + b2_ref[...]
    ea2 = ea + 0.5 * upd
    cw1 = cw1_ref[...]
    feat = (_dot(hs, cw1[0:H, :]) + _dot(hd, cw1[H:2 * H, :])
            + _dot(ea2, cw1[2 * H:3 * H, :]) + cb1_ref[...])
    out_ref[...] = _dot(jnp.maximum(feat, 0.0), cw2_ref[...]) + cb2_ref[...]


def _edge1_cls(hs, hd, ea1, w1, b1, w2, b2, cw1, cb1, cw2, cb2, n_cls):
    full = lambda shape: pl.BlockSpec(shape, lambda i: (0, 0))
    eb = pl.BlockSpec((BE, H), lambda i: (i, 0))
    ebw = pl.BlockSpec((BE, W128), lambda i: (i, 0))
    return pl.pallas_call(
        _edge1_cls_body,
        grid=(N_EBLK,),
        in_specs=[
            ebw, ebw, eb,
            full((3 * H, H)), full((1, H)), full((H, H)), full((1, H)),
            full((3 * H, H)), full((1, H)), full((H, n_cls)), full((1, n_cls)),
        ],
        out_specs=pl.BlockSpec((BE, n_cls), lambda i: (i, 0)),
        out_shape=jax.ShapeDtypeStruct((E_PAD, n_cls), _f32),
    )(hs, hd, ea1, w1, b1, w2, b2, cw1, cb1, cw2, cb2)


def kernel(x, edge_index, edge_attr, enc_W, enc_b, node_W, node_b,
           edge_W, edge_b, gin_W1, gin_b1, gin_W2, gin_b2, eps,
           emlp_W1, emlp_b1, emlp_W2, emlp_b2,
           cls_W1, cls_b1, cls_W2, cls_b2):
    n_cls = cls_W2.shape[1]
    src = edge_index[0].astype(jnp.int32)
    dst = edge_index[1].astype(jnp.int32)
    srcp = jnp.zeros((E_PAD,), jnp.int32).at[:E].set(src).reshape(NW, KPW, CHUNK)
    dstp = (jnp.full((E_PAD,), N, jnp.int32).at[:E].set(dst)
            .reshape(NW, KPW, CHUNK))
    attr_p = jnp.zeros((E_PAD, C), _f32).at[:E].set(edge_attr)
    x_pad = jnp.zeros((N_PAD, D), _f32).at[:N].set(x)
    encb_flat = enc_b.reshape(1, C * H)
    edgeb2 = edge_b.reshape(1, H)
    nodeb2 = node_b.reshape(1, H)

    gather1 = _make_gather(1)
    gather2 = _make_gather(2)
    scatter = _make_scatter()

    h0 = _node_proj(x_pad, node_W, nodeb2)
    hs0 = gather1(h0, srcp)
    msg0 = _msg0(hs0, attr_p, enc_W, encb_flat, edge_W, edgeb2)
    ag0 = scatter(msg0, dstp).reshape(NW, Q, 128, ACC_COLS)
    h1 = _node_update(h0, ag0, gin_W1[0], gin_b1[0:1], gin_W2[0],
                      gin_b2[0:1], eps[0].reshape(1, 1))
    hs1, hd1 = gather2(h1, srcp, dstp)
    ea1, msg1 = _edge0(hs1, hd1, attr_p, enc_W, encb_flat, edge_W, edgeb2,
                       emlp_W1[0], emlp_b1[0:1], emlp_W2[0], emlp_b2[0:1])
    ag1 = scatter(msg1, dstp).reshape(NW, Q, 128, ACC_COLS)
    h2 = _node_update(h1, ag1, gin_W1[1], gin_b1[1:2], gin_W2[1],
                      gin_b2[1:2], eps[1].reshape(1, 1))
    hs2, hd2 = gather2(h2, srcp, dstp)
    logits = _edge1_cls(hs2, hd2, ea1, emlp_W1[1], emlp_b1[1:2],
                        emlp_W2[1], emlp_b2[1:2],
                        cls_W1, cls_b1.reshape(1, H), cls_W2,
                        cls_b2.reshape(1, n_cls), n_cls)
    return logits[:E]
